# baseline (speedup 1.0000x reference)
import functools

import jax
import jax.numpy as jnp
from jax import lax
from jax.experimental import pallas as pl
from jax.experimental.pallas import tpu as pltpu
from jax.experimental.pallas import tpu_sc as plsc

_N = 4096
_E = 131072
_NC = 2
_NT = 16
_WORDS = _N * _N
_HALF = _WORDS // _NC
_WPT = _HALF // _NT
_ZB = 65536
_EPT = _E // _NT
_CH = 128
_NCHUNK = _EPT // _CH
_DPT = _N // _NC // _NT
_DCHUNK = _DPT // _CH if _DPT >= _CH else 1


def _build_adj(src, dst):
    mesh = plsc.VectorSubcoreMesh(core_axis_name="c", subcore_axis_name="s")

    @functools.partial(
        pl.kernel, mesh=mesh,
        out_type=jax.ShapeDtypeStruct((_WORDS,), jnp.float32),
        scratch_types=[
            pltpu.VMEM((_ZB,), jnp.float32),
            pltpu.VMEM((_CH,), jnp.float32),
            pltpu.VMEM((_NCHUNK, _CH), jnp.int32),
            pltpu.VMEM((_DCHUNK, _CH), jnp.int32),
            pltpu.VMEM((_EPT,), jnp.int32),
            pltpu.VMEM((_EPT,), jnp.int32),
            pltpu.SemaphoreType.DMA,
        ],
    )
    def adj_kernel(src_hbm, dst_hbm, out_hbm, zer_v, one_v, eidx_v, didx_v,
                   src_v, dst_v, sem):
        cid = lax.axis_index("c")
        tid = lax.axis_index("s")
        lanes = lax.iota(jnp.int32, 16)
        half_lo = cid * _HALF
        dump = (cid * (_N // _NC)) * (_N + 1)

        def fill_zeros(i, _):
            zer_v[pl.ds(i * 16, 16)] = jnp.zeros((16,), jnp.float32)
            return 0
        lax.fori_loop(0, _ZB // 16, fill_zeros, 0)

        def fill_ones(i, _):
            one_v[pl.ds(i * 16, 16)] = jnp.ones((16,), jnp.float32)
            return 0
        lax.fori_loop(0, _CH // 16, fill_ones, 0)

        pltpu.sync_copy(src_hbm.at[pl.ds(tid * _EPT, _EPT)], src_v)
        pltpu.sync_copy(dst_hbm.at[pl.ds(tid * _EPT, _EPT)], dst_v)

        def fill_eidx(i, _):
            s16 = src_v[pl.ds(i * 16, 16)]
            d16 = dst_v[pl.ds(i * 16, 16)]
            flat = s16 * _N + d16
            mine = (flat >= half_lo) & (flat < half_lo + _HALF)
            r = i // (_CH // 16)
            c = (i % (_CH // 16)) * 16
            eidx_v[r, pl.ds(c, 16)] = jnp.where(mine, flat, dump)
            return 0
        lax.fori_loop(0, _EPT // 16, fill_eidx, 0)

        dbase = cid * (_N // _NC) + tid * _DPT

        def fill_didx(i, _):
            r = i // (_CH // 16)
            c = (i % (_CH // 16)) * 16
            didx_v[r, pl.ds(c, 16)] = (dbase + i * 16 + lanes) * (_N + 1)
            return 0
        lax.fori_loop(0, _DPT // 16, fill_didx, 0)

        zbase = half_lo + tid * _WPT

        def zero_start(j, _):
            pltpu.async_copy(zer_v, out_hbm.at[pl.ds(zbase + j * _ZB, _ZB)],
                             sem)
            return 0
        lax.fori_loop(0, _WPT // _ZB, zero_start, 0)

        def zero_drain(j, _):
            pltpu.make_async_copy(
                zer_v, out_hbm.at[pl.ds(zbase + j * _ZB, _ZB)], sem).wait()
            return 0
        lax.fori_loop(0, _WPT // _ZB, zero_drain, 0)

        plsc.subcore_barrier()

        def scat_diag(r, _):
            pltpu.async_copy(one_v, out_hbm.at[didx_v.at[r]], sem)
            return 0
        lax.fori_loop(0, _DCHUNK, scat_diag, 0)

        def scat_edge(r, _):
            pltpu.async_copy(one_v, out_hbm.at[eidx_v.at[r]], sem)
            return 0
        lax.fori_loop(0, _NCHUNK, scat_edge, 0)

        def scat_drain(r, _):
            pltpu.make_async_copy(one_v, out_hbm.at[didx_v.at[0]], sem).wait()
            return 0
        lax.fori_loop(0, _NCHUNK + _DCHUNK, scat_drain, 0)

    return adj_kernel(src, dst)


def _mm_bin(a, b, out_dtype, bm=1024, bn=1024):
    grid = (_N // bm, _N // bn)

    def body(a_ref, b_ref, o_ref):
        o_ref[...] = (jnp.dot(a_ref[...], b_ref[...],
                              preferred_element_type=jnp.float32)
                      > 0).astype(out_dtype)

    return pl.pallas_call(
        body,
        grid=grid,
        in_specs=[
            pl.BlockSpec((bm, _N), lambda i, j: (i, 0)),
            pl.BlockSpec((_N, bn), lambda i, j: (0, j)),
        ],
        out_specs=pl.BlockSpec((bm, bn), lambda i, j: (i, j)),
        out_shape=jax.ShapeDtypeStruct((_N, _N), out_dtype),
        compiler_params=pltpu.CompilerParams(
            dimension_semantics=("parallel", "parallel")),
    )(a, b)


def kernel(edge_index, num_nodes):
    ei = edge_index.astype(jnp.int32)
    adj_flat = _build_adj(ei[0], ei[1])
    adj = adj_flat.reshape(_N, _N).astype(jnp.bfloat16)
    hop2 = _mm_bin(adj, adj, jnp.bfloat16)
    return _mm_bin(hop2, adj, jnp.float32)


# speedup vs baseline: 14.4219x; 14.4219x over previous
import functools

import jax
import jax.numpy as jnp
from jax import lax
from jax.experimental import pallas as pl
from jax.experimental.pallas import tpu as pltpu
from jax.experimental.pallas import tpu_sc as plsc

_N = 4096
_E = 131072
_NC = 2
_NT = 16
_WORDS = _N * _N
_HALF = _WORDS // _NC
_WPT = _HALF // _NT
_ZB = 65536
_EPT = _E // _NT
_CH = 128
_NCHUNK = _EPT // _CH
_DPT = _N // _NC // _NT
_DCHUNK = _DPT // _CH if _DPT >= _CH else 1


def _build_adj(src, dst):
    mesh = plsc.VectorSubcoreMesh(core_axis_name="c", subcore_axis_name="s")

    @functools.partial(
        pl.kernel, mesh=mesh,
        out_type=jax.ShapeDtypeStruct((_WORDS,), jnp.float32),
        scratch_types=[
            pltpu.VMEM((_ZB,), jnp.float32),
            pltpu.VMEM((_CH,), jnp.float32),
            pltpu.VMEM((_NCHUNK, _CH), jnp.int32),
            pltpu.VMEM((_DCHUNK, _CH), jnp.int32),
            pltpu.VMEM((_EPT,), jnp.int32),
            pltpu.VMEM((_EPT,), jnp.int32),
            pltpu.SemaphoreType.DMA,
        ],
    )
    def adj_kernel(src_hbm, dst_hbm, out_hbm, zer_v, one_v, eidx_v, didx_v,
                   src_v, dst_v, sem):
        cid = lax.axis_index("c")
        tid = lax.axis_index("s")
        lanes = lax.iota(jnp.int32, 16)
        half_lo = cid * _HALF
        drow0 = cid * (_N // _NC)

        def fill_zeros(i, _):
            zer_v[pl.ds(i * 16, 16)] = jnp.zeros((16,), jnp.float32)
            return 0
        lax.fori_loop(0, _ZB // 16, fill_zeros, 0)

        def fill_ones(i, _):
            one_v[pl.ds(i * 16, 16)] = jnp.ones((16,), jnp.float32)
            return 0
        lax.fori_loop(0, _CH // 16, fill_ones, 0)

        pltpu.sync_copy(src_hbm.at[pl.ds(tid * _EPT, _EPT)], src_v)
        pltpu.sync_copy(dst_hbm.at[pl.ds(tid * _EPT, _EPT)], dst_v)

        def fill_eidx(i, _):
            s16 = src_v[pl.ds(i * 16, 16)]
            d16 = dst_v[pl.ds(i * 16, 16)]
            flat = s16 * _N + d16
            mine = (flat >= half_lo) & (flat < half_lo + _HALF)
            dump = (drow0 + (d16 & (_N // _NC - 1))) * (_N + 1)
            r = i // (_CH // 16)
            c = (i % (_CH // 16)) * 16
            eidx_v[r, pl.ds(c, 16)] = jnp.where(mine, flat, dump)
            return 0
        lax.fori_loop(0, _EPT // 16, fill_eidx, 0)

        dbase = cid * (_N // _NC) + tid * _DPT

        def fill_didx(i, _):
            r = i // (_CH // 16)
            c = (i % (_CH // 16)) * 16
            didx_v[r, pl.ds(c, 16)] = (dbase + i * 16 + lanes) * (_N + 1)
            return 0
        lax.fori_loop(0, _DPT // 16, fill_didx, 0)

        zbase = half_lo + tid * _WPT

        def zero_start(j, _):
            pltpu.async_copy(zer_v, out_hbm.at[pl.ds(zbase + j * _ZB, _ZB)],
                             sem)
            return 0
        lax.fori_loop(0, _WPT // _ZB, zero_start, 0)

        def zero_drain(j, _):
            pltpu.make_async_copy(
                zer_v, out_hbm.at[pl.ds(zbase + j * _ZB, _ZB)], sem).wait()
            return 0
        lax.fori_loop(0, _WPT // _ZB, zero_drain, 0)

        plsc.subcore_barrier()

        def scat_diag(r, _):
            pltpu.async_copy(one_v, out_hbm.at[didx_v.at[r]], sem)
            return 0
        lax.fori_loop(0, _DCHUNK, scat_diag, 0)

        def scat_edge(r, _):
            pltpu.async_copy(one_v, out_hbm.at[eidx_v.at[r]], sem)
            return 0
        lax.fori_loop(0, _NCHUNK, scat_edge, 0)

        def scat_drain(r, _):
            pltpu.make_async_copy(one_v, out_hbm.at[didx_v.at[0]], sem).wait()
            return 0
        lax.fori_loop(0, _NCHUNK + _DCHUNK, scat_drain, 0)

    return adj_kernel(src, dst)


def _mm_bin(a, b, out_dtype, bm=1024, bn=1024):
    grid = (_N // bm, _N // bn)

    def body(a_ref, b_ref, o_ref):
        o_ref[...] = (jnp.dot(a_ref[...], b_ref[...],
                              preferred_element_type=jnp.float32)
                      > 0).astype(out_dtype)

    return pl.pallas_call(
        body,
        grid=grid,
        in_specs=[
            pl.BlockSpec((bm, _N), lambda i, j: (i, 0)),
            pl.BlockSpec((_N, bn), lambda i, j: (0, j)),
        ],
        out_specs=pl.BlockSpec((bm, bn), lambda i, j: (i, j)),
        out_shape=jax.ShapeDtypeStruct((_N, _N), out_dtype),
        compiler_params=pltpu.CompilerParams(
            dimension_semantics=("parallel", "parallel")),
    )(a, b)


def kernel(edge_index, num_nodes):
    ei = edge_index.astype(jnp.int32)
    adj_flat = _build_adj(ei[0], ei[1])
    adj = adj_flat.reshape(_N, _N).astype(jnp.bfloat16)
    hop2 = _mm_bin(adj, adj, jnp.bfloat16)
    return _mm_bin(hop2, adj, jnp.float32)


# speedup vs baseline: 18.1045x; 1.2554x over previous
import functools

import jax
import jax.numpy as jnp
from jax import lax
from jax.experimental import pallas as pl
from jax.experimental.pallas import tpu as pltpu
from jax.experimental.pallas import tpu_sc as plsc

_N = 4096
_E = 131072
_NT = 16
_WORDS = _N * _N
_WPT = _WORDS // _NT
_ZB = 16384
_EPT = _E // _NT
_CH = 128
_NCHUNK = _EPT // _CH
_DPT = _N // _NT
_DCHUNK = _DPT // _CH


def _build_adj(src, dst):
    mesh = plsc.VectorSubcoreMesh(
        core_axis_name="c", subcore_axis_name="s", num_cores=1)

    @functools.partial(
        pl.kernel, mesh=mesh,
        out_type=jax.ShapeDtypeStruct((_WORDS,), jnp.float32),
        scratch_types=[
            pltpu.VMEM((_ZB,), jnp.float32),
            pltpu.VMEM((_CH,), jnp.float32),
            pltpu.VMEM((_NCHUNK, _CH), jnp.int32),
            pltpu.VMEM((_DCHUNK, _CH), jnp.int32),
            pltpu.VMEM((_EPT,), jnp.int32),
            pltpu.VMEM((_EPT,), jnp.int32),
            pltpu.SemaphoreType.DMA,
        ],
    )
    def adj_kernel(src_hbm, dst_hbm, out_hbm, zer_v, one_v, eidx_v, didx_v,
                   src_v, dst_v, sem):
        tid = lax.axis_index("s")
        lanes = lax.iota(jnp.int32, 16)

        def fill_zeros(i, _):
            zer_v[pl.ds(i * 16, 16)] = jnp.zeros((16,), jnp.float32)
            return 0
        lax.fori_loop(0, _ZB // 16, fill_zeros, 0)

        def fill_ones(i, _):
            one_v[pl.ds(i * 16, 16)] = jnp.ones((16,), jnp.float32)
            return 0
        lax.fori_loop(0, _CH // 16, fill_ones, 0)

        pltpu.sync_copy(src_hbm.at[pl.ds(tid * _EPT, _EPT)], src_v)
        pltpu.sync_copy(dst_hbm.at[pl.ds(tid * _EPT, _EPT)], dst_v)

        def fill_eidx(i, _):
            s16 = src_v[pl.ds(i * 16, 16)]
            d16 = dst_v[pl.ds(i * 16, 16)]
            r = i // (_CH // 16)
            c = (i % (_CH // 16)) * 16
            eidx_v[r, pl.ds(c, 16)] = s16 * _N + d16
            return 0
        lax.fori_loop(0, _EPT // 16, fill_eidx, 0)

        dbase = tid * _DPT

        def fill_didx(i, _):
            r = i // (_CH // 16)
            c = (i % (_CH // 16)) * 16
            didx_v[r, pl.ds(c, 16)] = (dbase + i * 16 + lanes) * (_N + 1)
            return 0
        lax.fori_loop(0, _DPT // 16, fill_didx, 0)

        zbase = tid * _WPT

        def zero_start(j, _):
            pltpu.async_copy(zer_v, out_hbm.at[pl.ds(zbase + j * _ZB, _ZB)],
                             sem)
            return 0
        lax.fori_loop(0, _WPT // _ZB, zero_start, 0)

        def zero_drain(j, _):
            pltpu.make_async_copy(
                zer_v, out_hbm.at[pl.ds(zbase + j * _ZB, _ZB)], sem).wait()
            return 0
        lax.fori_loop(0, _WPT // _ZB, zero_drain, 0)

        plsc.subcore_barrier()

        def scat_diag(r, _):
            pltpu.async_copy(one_v, out_hbm.at[didx_v.at[r]], sem)
            return 0
        lax.fori_loop(0, _DCHUNK, scat_diag, 0)

        def scat_edge(r, _):
            pltpu.async_copy(one_v, out_hbm.at[eidx_v.at[r]], sem)
            return 0
        lax.fori_loop(0, _NCHUNK, scat_edge, 0)

        def scat_drain(r, _):
            pltpu.make_async_copy(one_v, out_hbm.at[didx_v.at[0]],
                                  sem).wait()
            return 0
        lax.fori_loop(0, _NCHUNK + _DCHUNK, scat_drain, 0)

    return adj_kernel(src, dst)


def _mm_bin(a, b, out_dtype, bm=1024, bn=1024):
    grid = (_N // bm, _N // bn)

    def body(a_ref, b_ref, o_ref):
        o_ref[...] = (jnp.dot(a_ref[...], b_ref[...],
                              preferred_element_type=jnp.float32)
                      > 0).astype(out_dtype)

    return pl.pallas_call(
        body,
        grid=grid,
        in_specs=[
            pl.BlockSpec((bm, _N), lambda i, j: (i, 0)),
            pl.BlockSpec((_N, bn), lambda i, j: (0, j)),
        ],
        out_specs=pl.BlockSpec((bm, bn), lambda i, j: (i, j)),
        out_shape=jax.ShapeDtypeStruct((_N, _N), out_dtype),
        compiler_params=pltpu.CompilerParams(
            dimension_semantics=("parallel", "parallel")),
    )(a, b)


def kernel(edge_index, num_nodes):
    ei = edge_index.astype(jnp.int32)
    adj_flat = _build_adj(ei[0], ei[1])
    adj = adj_flat.reshape(_N, _N).astype(jnp.bfloat16)
    hop2 = _mm_bin(adj, adj, jnp.bfloat16)
    return _mm_bin(hop2, adj, jnp.float32)


# speedup vs baseline: 18.5873x; 1.0267x over previous
import functools

import jax
import jax.numpy as jnp
from jax import lax
from jax.experimental import pallas as pl
from jax.experimental.pallas import tpu as pltpu
from jax.experimental.pallas import tpu_sc as plsc

_N = 4096
_E = 131072
_NW = 32
_WORDS = _N * _N
_WPT = _WORDS // _NW
_ZB = 16384
_EPT = _E // _NW
_CH = 128
_NCHUNK = _EPT // _CH
_DPT = _N // _NW
_DCHUNK = 1


def _build_adj(src, dst):
    mesh = plsc.VectorSubcoreMesh(core_axis_name="c", subcore_axis_name="s")

    @functools.partial(
        pl.kernel, mesh=mesh,
        out_type=jax.ShapeDtypeStruct((_WORDS,), jnp.float32),
        scratch_types=[
            pltpu.VMEM((_ZB,), jnp.float32),
            pltpu.VMEM((_CH,), jnp.float32),
            pltpu.VMEM((_NCHUNK, _CH), jnp.int32),
            pltpu.VMEM((_DCHUNK, _CH), jnp.int32),
            pltpu.VMEM((_EPT,), jnp.int32),
            pltpu.VMEM((_EPT,), jnp.int32),
            pltpu.SemaphoreType.REGULAR,
            pltpu.SemaphoreType.DMA,
        ],
    )
    def adj_kernel(src_hbm, dst_hbm, out_hbm, zer_v, one_v, eidx_v, didx_v,
                   src_v, dst_v, xsem, sem):
        cid = lax.axis_index("c")
        tid = lax.axis_index("s")
        woff = cid * 16 + tid
        lanes = lax.iota(jnp.int32, 16)

        def fill_zeros(i, _):
            zer_v[pl.ds(i * 16, 16)] = jnp.zeros((16,), jnp.float32)
            return 0
        lax.fori_loop(0, _ZB // 16, fill_zeros, 0)

        def fill_ones(i, _):
            one_v[pl.ds(i * 16, 16)] = jnp.ones((16,), jnp.float32)
            return 0
        lax.fori_loop(0, _CH // 16, fill_ones, 0)

        pltpu.sync_copy(src_hbm.at[pl.ds(woff * _EPT, _EPT)], src_v)
        pltpu.sync_copy(dst_hbm.at[pl.ds(woff * _EPT, _EPT)], dst_v)

        def fill_eidx(i, _):
            s16 = src_v[pl.ds(i * 16, 16)]
            d16 = dst_v[pl.ds(i * 16, 16)]
            r = i // (_CH // 16)
            c = (i % (_CH // 16)) * 16
            eidx_v[r, pl.ds(c, 16)] = s16 * _N + d16
            return 0
        lax.fori_loop(0, _EPT // 16, fill_eidx, 0)

        dbase = woff * _DPT

        def fill_didx(i, _):
            didx_v[0, pl.ds(i * 16, 16)] = (dbase + i * 16 + lanes) * (_N + 1)
            return 0
        lax.fori_loop(0, _DPT // 16, fill_didx, 0)

        zbase = woff * _WPT

        def zero_start(j, _):
            pltpu.async_copy(zer_v, out_hbm.at[pl.ds(zbase + j * _ZB, _ZB)],
                             sem)
            return 0
        lax.fori_loop(0, _WPT // _ZB, zero_start, 0)

        def zero_drain(j, _):
            pltpu.make_async_copy(
                zer_v, out_hbm.at[pl.ds(zbase + j * _ZB, _ZB)], sem).wait()
            return 0
        lax.fori_loop(0, _WPT // _ZB, zero_drain, 0)

        plsc.subcore_barrier()

        @pl.when(tid == 0)
        def _handshake():
            pltpu.semaphore_signal(xsem, 1, core_index=1 - cid)
            pltpu.semaphore_wait(xsem, 1)

        plsc.subcore_barrier()

        pltpu.async_copy(one_v, out_hbm.at[didx_v.at[0]], sem)

        def scat_edge(r, _):
            pltpu.async_copy(one_v, out_hbm.at[eidx_v.at[r]], sem)
            return 0
        lax.fori_loop(0, _NCHUNK, scat_edge, 0)

        def scat_drain(r, _):
            pltpu.make_async_copy(one_v, out_hbm.at[didx_v.at[0]],
                                  sem).wait()
            return 0
        lax.fori_loop(0, _NCHUNK + _DCHUNK, scat_drain, 0)

    return adj_kernel(src, dst)


def _mm_bin(a, b, out_dtype, bm=1024, bn=1024):
    grid = (_N // bm, _N // bn)

    def body(a_ref, b_ref, o_ref):
        o_ref[...] = (jnp.dot(a_ref[...], b_ref[...],
                              preferred_element_type=jnp.float32)
                      > 0).astype(out_dtype)

    return pl.pallas_call(
        body,
        grid=grid,
        in_specs=[
            pl.BlockSpec((bm, _N), lambda i, j: (i, 0)),
            pl.BlockSpec((_N, bn), lambda i, j: (0, j)),
        ],
        out_specs=pl.BlockSpec((bm, bn), lambda i, j: (i, j)),
        out_shape=jax.ShapeDtypeStruct((_N, _N), out_dtype),
        compiler_params=pltpu.CompilerParams(
            dimension_semantics=("parallel", "parallel")),
    )(a, b)


def kernel(edge_index, num_nodes):
    ei = edge_index.astype(jnp.int32)
    adj_flat = _build_adj(ei[0], ei[1])
    adj = adj_flat.reshape(_N, _N).astype(jnp.bfloat16)
    hop2 = _mm_bin(adj, adj, jnp.bfloat16)
    return _mm_bin(hop2, adj, jnp.float32)
